# Initial kernel scaffold; baseline (speedup 1.0000x reference)
import jax
import jax.numpy as jnp
from jax.experimental import pallas as pl


def kernel(x, edge_index, W, b):
    raise NotImplementedError("write your pallas kernel here")


# speedup vs baseline: 17.5341x; 17.5341x over previous
import functools

import jax
import jax.numpy as jnp
from jax import lax
from jax.experimental import pallas as pl
from jax.experimental.pallas import tpu as pltpu
from jax.experimental.pallas import tpu_sc as plsc

NC = 2
NS = 16
NW = NC * NS
K = 80

_MESH = dict(core_axis_name="c", subcore_axis_name="s")


def _deg_kernel(n_nodes: int, n_edges: int):
    epw = n_edges // NW
    iters = epw // K
    nchunks = n_nodes // K

    @functools.partial(
        pl.kernel,
        out_type=jax.ShapeDtypeStruct((NC * n_nodes,), jnp.float32),
        mesh=plsc.VectorSubcoreMesh(**_MESH),
        scratch_types=[
            pltpu.VMEM((K,), jnp.int32),
            pltpu.VMEM((K,), jnp.float32),
            pltpu.VMEM((K,), jnp.float32),
            pltpu.VMEM_SHARED((n_nodes,), jnp.float32),
        ],
    )
    def deg(row_hbm, out_hbm, idx_v, ones_v, zb_v, deg_sh):
        c = lax.axis_index("c")
        s = lax.axis_index("s")
        w = s * NC + c
        for j in range(K // 16):
            ones_v[pl.ds(j * 16, 16)] = jnp.ones((16,), jnp.float32)
            zb_v[pl.ds(j * 16, 16)] = jnp.zeros((16,), jnp.float32)

        def zbody(i, carry):
            off = pl.multiple_of((s + i * NS) * K, 8)
            pltpu.sync_copy(zb_v, deg_sh.at[pl.ds(off, K)])
            return carry

        lax.fori_loop(0, (nchunks - s + NS - 1) // NS, zbody, 0)
        plsc.subcore_barrier()

        def body(i, carry):
            off = pl.multiple_of(w * epw + i * K, 8)
            pltpu.sync_copy(row_hbm.at[pl.ds(off, K)], idx_v)
            pltpu.sync_copy(ones_v, deg_sh.at[idx_v], add=True)
            return carry

        lax.fori_loop(0, iters, body, 0)
        plsc.subcore_barrier()

        def obody(i, carry):
            off = pl.multiple_of((s + i * NS) * K, 8)
            pltpu.sync_copy(deg_sh.at[pl.ds(off, K)], zb_v)
            oo = pl.multiple_of(c * n_nodes + (s + i * NS) * K, 8)
            pltpu.sync_copy(zb_v, out_hbm.at[pl.ds(oo, K)])
            return carry

        lax.fori_loop(0, (nchunks - s + NS - 1) // NS, obody, 0)

    return deg


def _aggr_kernel(n_nodes: int, n_edges: int, d: int):
    epw = n_edges // NW
    iters = epw // K
    nchunks = n_nodes // K

    @functools.partial(
        pl.kernel,
        out_type=jax.ShapeDtypeStruct((NC, n_nodes, d), jnp.float32),
        mesh=plsc.VectorSubcoreMesh(**_MESH),
        scratch_types=[
            pltpu.VMEM((K,), jnp.int32),
            pltpu.VMEM((K,), jnp.int32),
            pltpu.VMEM((K, d), jnp.float32),
            pltpu.VMEM((K, d), jnp.float32),
            pltpu.VMEM_SHARED((n_nodes, d), jnp.float32),
            pltpu.SemaphoreType.DMA,
        ],
    )
    def aggr(col_hbm, row_hbm, g_hbm, out_hbm,
             col_v, row_v, rows_v, zb_v, acc_sh, sem):
        c = lax.axis_index("c")
        s = lax.axis_index("s")
        w = s * NC + c

        def zrow(i, carry):
            def zcol(j, carry2):
                zb_v[i, pl.ds(j * 16, 16)] = jnp.zeros((16,), jnp.float32)
                return carry2
            return lax.fori_loop(0, d // 16, zcol, carry)

        lax.fori_loop(0, K, zrow, 0)

        def zbody(i, carry):
            off = pl.multiple_of((s + i * NS) * K, 8)
            pltpu.sync_copy(zb_v, acc_sh.at[pl.ds(off, K)])
            return carry

        lax.fori_loop(0, (nchunks - s + NS - 1) // NS, zbody, 0)
        plsc.subcore_barrier()

        def body(i, carry):
            off = pl.multiple_of(w * epw + i * K, 8)
            pltpu.sync_copy(col_hbm.at[pl.ds(off, K)], col_v)
            pltpu.sync_copy(row_hbm.at[pl.ds(off, K)], row_v)
            pltpu.async_copy(g_hbm.at[col_v], rows_v, sem).wait()
            pltpu.sync_copy(rows_v, acc_sh.at[row_v], add=True)
            return carry

        lax.fori_loop(0, iters, body, 0)
        plsc.subcore_barrier()

        def obody(i, carry):
            off = pl.multiple_of((s + i * NS) * K, 8)
            pltpu.sync_copy(acc_sh.at[pl.ds(off, K)], zb_v)
            pltpu.sync_copy(zb_v, out_hbm.at[c, pl.ds(off, K)])
            return carry

        lax.fori_loop(0, (nchunks - s + NS - 1) // NS, obody, 0)

    return aggr


def _tc1_call(x, wt, b2, degt):
    n, d_in = x.shape
    d_out = wt.shape[1]
    br = 2000
    grid = (n // br,)

    def tc1(x_ref, wt_ref, b_ref, degt_ref, g_ref, dis_ref):
        deg = degt_ref[:, 0] + degt_ref[:, 1] + 1.0
        dis = lax.rsqrt(deg)
        h = jnp.dot(x_ref[...], wt_ref[...],
                    preferred_element_type=jnp.float32) + b_ref[...]
        g_ref[...] = h * dis[:, None]
        dis_ref[...] = dis[:, None]

    return pl.pallas_call(
        tc1,
        grid=grid,
        in_specs=[
            pl.BlockSpec((br, d_in), lambda i: (i, 0)),
            pl.BlockSpec((d_in, d_out), lambda i: (0, 0)),
            pl.BlockSpec((1, d_out), lambda i: (0, 0)),
            pl.BlockSpec((br, 2), lambda i: (i, 0)),
        ],
        out_specs=[
            pl.BlockSpec((br, d_out), lambda i: (i, 0)),
            pl.BlockSpec((br, 1), lambda i: (i, 0)),
        ],
        out_shape=[
            jax.ShapeDtypeStruct((n, d_out), jnp.float32),
            jax.ShapeDtypeStruct((n, 1), jnp.float32),
        ],
    )(x, wt, b2, degt)


def _tc2_call(part, g, dis):
    n, d = g.shape
    br = 2000
    grid = (n // br,)

    def tc2(p_ref, g_ref, dis_ref, out_ref):
        out_ref[...] = dis_ref[...] * (p_ref[0] + p_ref[1] + g_ref[...])

    return pl.pallas_call(
        tc2,
        grid=grid,
        in_specs=[
            pl.BlockSpec((NC, br, d), lambda i: (0, i, 0)),
            pl.BlockSpec((br, d), lambda i: (i, 0)),
            pl.BlockSpec((br, 1), lambda i: (i, 0)),
        ],
        out_specs=pl.BlockSpec((br, d), lambda i: (i, 0)),
        out_shape=jax.ShapeDtypeStruct((n, d), jnp.float32),
    )(part, g, dis)


def kernel(x, edge_index, W, b):
    n, d_in = x.shape
    d_out = W.shape[0]
    e = edge_index.shape[1]
    assert e % (NW * K) == 0 and n % NS == 0

    row = edge_index[0]
    col = edge_index[1]

    degp = _deg_kernel(n, e)(row)
    degt = degp.reshape(NC, n).T
    g, dis = _tc1_call(x, W.T, b.reshape(1, -1), degt)
    part = _aggr_kernel(n, e, d_out)(col, row, g)
    return _tc2_call(part, g, dis)
